# speedup vs baseline: 2.2337x; 2.2337x over previous
import jax
import jax.numpy as jnp
from jax.experimental import pallas as pl
from jax.experimental.pallas import tpu as pltpu

_NUM_HEADS = 8
_MAX_SEQ_LEN = 4096
_HEAD_DIM = 128
_TOTAL_SEQ_LEN = 32768
_BATCH = 16

_BLK_T = 512
_NUM_BLK = _TOTAL_SEQ_LEN // _BLK_T


def _tc_body(offs_ref, o_ref, do_ref, out_ref):
    i = pl.program_id(0)
    prod = o_ref[...] * do_ref[...]
    red = jnp.sum(prod, axis=-1)

    t = i * _BLK_T + jax.lax.broadcasted_iota(jnp.int32, (_NUM_HEADS, _BLK_T), 1)
    valid = jnp.zeros((_NUM_HEADS, _BLK_T), dtype=jnp.bool_)
    for b in range(_BATCH):
        begin = offs_ref[b]
        end = offs_ref[b + 1]
        stop = jnp.minimum(end, begin + _MAX_SEQ_LEN)
        valid = valid | ((t >= begin) & (t < stop))
    out_ref[...] = jnp.where(valid, red, 0.0)


def kernel(o, do, o_offset):
    grid_spec = pltpu.PrefetchScalarGridSpec(
        num_scalar_prefetch=1,
        grid=(_NUM_BLK,),
        in_specs=[
            pl.BlockSpec((_NUM_HEADS, _BLK_T, _HEAD_DIM), lambda i, offs: (0, i, 0)),
            pl.BlockSpec((_NUM_HEADS, _BLK_T, _HEAD_DIM), lambda i, offs: (0, i, 0)),
        ],
        out_specs=pl.BlockSpec((_NUM_HEADS, _BLK_T), lambda i, offs: (0, i)),
    )
    return pl.pallas_call(
        _tc_body,
        grid_spec=grid_spec,
        out_shape=jax.ShapeDtypeStruct((_NUM_HEADS, _TOTAL_SEQ_LEN), jnp.float32),
    )(o_offset, o, do)
